# baseline (speedup 1.0000x reference)
import functools

import jax
import jax.numpy as jnp
from jax import lax
from jax.experimental import pallas as pl
from jax.experimental.pallas import tpu as pltpu
from jax.experimental.pallas import tpu_sc as plsc

B = 8
D = 1024
V = 100000
K = 10
VT = 2048
NBLK = 49
VP = NBLK * VT
NBM = VT // 128
NE = VP // 128


def _logits_kernel(h_ref, w_ref, o_ref, m_ref):
    j = pl.program_id(0)
    logits = lax.dot_general(
        h_ref[...], w_ref[...], (((1,), (1,)), ((), ())),
        preferred_element_type=jnp.float32,
    )
    col = j * VT + lax.broadcasted_iota(jnp.int32, (B, VT), 1)
    lg = jnp.where(col < V, logits, -jnp.inf)
    o_ref[...] = lg
    bm = jnp.concatenate(
        [jnp.max(lg[:, c * 128:(c + 1) * 128], axis=1, keepdims=True)
         for c in range(NBM)], axis=1)
    m_ref[...] = bm.reshape(1, B, NBM)


def _merge16(tv, ti, v, iv):
    vs, ivs = plsc.sort_key_val(v, iv, descending=True)
    m = tv >= vs
    nv = jnp.where(m, tv, vs)
    ni = jnp.where(m, ti, ivs)
    out = plsc.sort_key_val(nv, ni)
    return out[0], out[1]


def _sc_topk_blend(lg_hbm, m_hbm, emb_hbm, out_hbm,
                   m_v, cand_v, rows_v, acc_v, sem):
    wid = lax.axis_index("s") * 2 + lax.axis_index("c")

    @pl.when(wid < B)
    def _():
        b = wid
        pltpu.sync_copy(m_hbm, m_v)
        iota = lax.broadcasted_iota(jnp.int32, (16,), 0)

        tv = jnp.full((16,), -jnp.inf, jnp.float32)
        ti = jnp.zeros((16,), jnp.int32)
        for j in range(NBLK):
            tv, ti = _merge16(tv, ti, m_v[j, b], j * NBM + iota)

        pltpu.async_copy(lg_hbm.at[b * NE + ti], cand_v, sem).wait()

        gv = jnp.full((16,), -jnp.inf, jnp.float32)
        gi = jnp.zeros((16,), jnp.int32)
        for l in range(16):
            base = ti[l] * 128

            def cbody(c, args, l=l, base=base):
                v = cand_v[l, pl.ds(c * 16, 16)]
                return _merge16(args[0], args[1], v, base + c * 16 + iota)

            gv, gi = lax.fori_loop(0, 128 // 16, cbody, (gv, gi))

        ev = jnp.exp(gv - jnp.full((16,), gv[15]))
        s = ev[16 - K]
        for j in range(16 - K + 1, 16):
            s = s + ev[j]
        wvec = ev / jnp.full((16,), s)
        wts = [wvec[j] for j in range(16 - K, 16)]

        pltpu.async_copy(emb_hbm.at[gi], rows_v, sem).wait()

        def fbody(c, carry):
            acc = jnp.zeros((16,), jnp.float32)
            for j in range(16 - K, 16):
                acc = acc + wts[j - (16 - K)] * rows_v[j, pl.ds(c * 16, 16)]
            acc_v[pl.ds(c * 16, 16)] = acc
            return carry

        lax.fori_loop(0, D // 16, fbody, 0)
        pltpu.sync_copy(acc_v, out_hbm.at[b])


def kernel(hidden_last, lm_head_w, emb_w):
    logits, bmax = pl.pallas_call(
        _logits_kernel,
        grid=(NBLK,),
        in_specs=[
            pl.BlockSpec((B, D), lambda j: (0, 0)),
            pl.BlockSpec((VT, D), lambda j: (j, 0)),
        ],
        out_specs=[
            pl.BlockSpec((B, VT), lambda j: (0, j)),
            pl.BlockSpec((1, B, NBM), lambda j: (j, 0, 0)),
        ],
        out_shape=[
            jax.ShapeDtypeStruct((B, VP), jnp.float32),
            jax.ShapeDtypeStruct((NBLK, B, NBM), jnp.float32),
        ],
    )(hidden_last, lm_head_w)

    logits2 = logits.reshape(B * NE, 128)

    mesh = plsc.VectorSubcoreMesh(core_axis_name="c", subcore_axis_name="s")
    blend = functools.partial(
        pl.kernel,
        mesh=mesh,
        compiler_params=pltpu.CompilerParams(needs_layout_passes=False),
        out_type=jax.ShapeDtypeStruct((B, D), jnp.float32),
        scratch_types=[
            pltpu.VMEM((NBLK, B, NBM), jnp.float32),
            pltpu.VMEM((16, 128), jnp.float32),
            pltpu.VMEM((16, D), jnp.float32),
            pltpu.VMEM((D,), jnp.float32),
            pltpu.SemaphoreType.DMA,
        ],
    )(_sc_topk_blend)
    return blend(logits2, bmax, emb_w)
